# baseline (speedup 1.0000x reference)
import functools

import jax
import jax.numpy as jnp
from jax import lax
from jax.experimental import pallas as pl
from jax.experimental.pallas import tpu as pltpu
from jax.experimental.pallas import tpu_sc as plsc

B, N, M, NS, C = 8, 4096, 1024, 32, 128
RADIUS = 0.12
R2 = RADIUS * RADIUS

NC, NSUB, L = 2, 16, 16
NW = NC * NSUB
MW = M // NW
NG = MW // L
KC = 8
TROWS = 136
NSLAB = 17
PU = 8
OUTC = 3 + C
MNS = M * NS
HALF = MNS // 2
SUBP = 2048
NSUB_B = HALF // SUBP


def _ball_query_group(pxyz_v, q_v, idx_v, grp, lanes):
    zeros = jnp.zeros((L,), jnp.int32)
    qsel = (grp * L + lanes) * 3
    qx = plsc.load_gather(q_v, [zeros, qsel])
    qy = plsc.load_gather(q_v, [zeros, qsel + 1])
    qz = plsc.load_gather(q_v, [zeros, qsel + 2])
    qoff = (grp * L + lanes) * NS
    plsc.store_scatter(idx_v, [qoff], zeros)

    def step(k, cnt):
        base = k * PU
        for j in range(PU):
            p = base + j
            pvec = jnp.full((L,), p, jnp.int32)
            px = plsc.load_gather(pxyz_v, [zeros, pvec])
            py = plsc.load_gather(pxyz_v, [zeros + 1, pvec])
            pz = plsc.load_gather(pxyz_v, [zeros + 2, pvec])
            dx = qx - px
            dy = qy - py
            dz = qz - pz
            d2 = dx * dx + dy * dy + dz * dz
            mask = d2 < R2
            wmask = mask & (cnt < NS)
            plsc.store_scatter(idx_v, [qoff + cnt], pvec, mask=wmask)
            cnt = cnt + mask.astype(jnp.int32)
        return cnt

    cnt = lax.fori_loop(0, N // PU, step, jnp.zeros((L,), jnp.int32))

    first = plsc.load_gather(idx_v, [qoff])
    for s in range(1, NS):
        cur = plsc.load_gather(idx_v, [qoff + s])
        sel = jnp.where(cnt > s, cur, first)
        plsc.store_scatter(idx_v, [qoff + s], sel)


def _bq_body(tab_hbm, q_hbm, idx_hbm, pxyz_v, q_v, idx_v):
    wid = lax.axis_index("s") * NC + lax.axis_index("c")
    lanes = lax.iota(jnp.int32, L)

    def per_batch(b, _):
        pltpu.sync_copy(tab_hbm.at[b, pl.ds(0, 3), :], pxyz_v)
        pltpu.sync_copy(q_hbm.at[b, wid], q_v)
        for grp in range(NG):
            _ball_query_group(pxyz_v, q_v, idx_v, grp, lanes)
        pltpu.sync_copy(idx_v, idx_hbm.at[b, 0, pl.ds(wid * MW * NS, MW * NS)])
        return 0

    lax.fori_loop(0, B, per_batch, 0)


def _gather_task(b, t, tab_hbm, q2_hbm, idx_hbm, out_hbm,
                 frows_v, q_v, iv_v, obuf_v, sems):
    slab = lax.shift_right_logical(t, 1)
    half = t & 1
    srow = pl.multiple_of(slab * KC, KC)
    hb = pl.multiple_of(half * HALF, 128)
    lanes = lax.iota(jnp.int32, L)

    pltpu.sync_copy(tab_hbm.at[b, pl.ds(srow, KC), :], frows_v)
    pltpu.sync_copy(q2_hbm.at[b, half], q_v)
    pltpu.sync_copy(idx_hbm.at[b, 0, pl.ds(hb, HALF)], iv_v)

    def fout(sub, slot, nrow, row0):
        return pltpu.make_async_copy(
            obuf_v.at[slot, pl.ds(0, nrow)],
            out_hbm.at[b, pl.ds(row0, nrow),
                       pl.ds(hb + sub * SUBP, SUBP)],
            sems.at[2 + slot])

    is_last = slab == NSLAB - 1

    for sub in range(NSUB_B):
        slot = sub % 2
        if sub >= 2:
            @pl.when(jnp.logical_not(is_last))
            def _():
                fout(sub - 2, slot, KC, srow).wait()
            @pl.when(is_last)
            def _():
                fout(sub - 2, slot, 3, 128).wait()

        def gath(tt, _):
            iv = iv_v[pl.ds(sub * SUBP + tt * L, L)]
            for c in range(KC):
                g = plsc.load_gather(frows_v,
                                     [jnp.full((L,), c, jnp.int32), iv])
                obuf_v[slot, c, pl.ds(tt * L, L)] = g
            return 0

        lax.fori_loop(0, SUBP // L, gath, 0)

        @pl.when(slab == 0)
        def _():
            def fix(tt, _):
                mloc3 = lax.shift_right_logical(sub * SUBP + tt * L + lanes,
                                                5) * 3
                for d in range(3):
                    qd = plsc.load_gather(
                        q_v, [jnp.zeros((L,), jnp.int32), mloc3 + d])
                    cur = obuf_v[slot, d, pl.ds(tt * L, L)]
                    obuf_v[slot, d, pl.ds(tt * L, L)] = cur - qd
                return 0
            lax.fori_loop(0, SUBP // L, fix, 0)

        @pl.when(jnp.logical_not(is_last))
        def _():
            fout(sub, slot, KC, srow).start()
        @pl.when(is_last)
        def _():
            fout(sub, slot, 3, 128).start()

    for sub in (NSUB_B - 2, NSUB_B - 1):
        @pl.when(jnp.logical_not(is_last))
        def _():
            fout(sub, sub % 2, KC, srow).wait()
        @pl.when(is_last)
        def _():
            fout(sub, sub % 2, 3, 128).wait()


def _gather_body(tab_hbm, q2_hbm, idx_hbm, out_hbm,
                 frows_v, q_v, iv_v, obuf_v, sems):
    wid = lax.axis_index("s") * NC + lax.axis_index("c")

    def per_batch(b, _):
        _gather_task(b, wid, tab_hbm, q2_hbm, idx_hbm, out_hbm,
                     frows_v, q_v, iv_v, obuf_v, sems)

        @pl.when(wid >= NW - 2)
        def _():
            _gather_task(b, wid + 2, tab_hbm, q2_hbm, idx_hbm, out_hbm,
                         frows_v, q_v, iv_v, obuf_v, sems)
        return 0

    lax.fori_loop(0, B, per_batch, 0)


_MESH = plsc.VectorSubcoreMesh(core_axis_name="c", subcore_axis_name="s")
_CP = pltpu.CompilerParams(needs_layout_passes=False)


@functools.partial(
    pl.kernel,
    out_type=jax.ShapeDtypeStruct((B, 1, MNS), jnp.int32),
    mesh=_MESH,
    scratch_types=[
        pltpu.VMEM((3, N), jnp.float32),
        pltpu.VMEM((1, MW * 3), jnp.float32),
        pltpu.VMEM((MW * NS,), jnp.int32),
    ],
    compiler_params=_CP,
)
def _bq_kernel(tab_hbm, q_hbm, idx_hbm, *scratch):
    _bq_body(tab_hbm, q_hbm, idx_hbm, *scratch)


@functools.partial(
    pl.kernel,
    out_type=jax.ShapeDtypeStruct((B, OUTC, MNS), jnp.float32),
    mesh=_MESH,
    scratch_types=[
        pltpu.VMEM((KC, N), jnp.float32),
        pltpu.VMEM((1, (M // 2) * 3), jnp.float32),
        pltpu.VMEM((HALF,), jnp.int32),
        pltpu.VMEM((2, KC, SUBP), jnp.float32),
        pltpu.SemaphoreType.DMA((4,)),
    ],
    compiler_params=_CP,
)
def _gather_kernel(tab_hbm, q2_hbm, idx_hbm, out_hbm, *scratch):
    _gather_body(tab_hbm, q2_hbm, idx_hbm, out_hbm, *scratch)


def kernel(xyz, new_xyz, features):
    xyz_t = jnp.transpose(xyz, (0, 2, 1))
    pad = jnp.zeros((B, TROWS - 3 - C, N), jnp.float32)
    tab = jnp.concatenate([xyz_t, features, pad], axis=1)
    q = new_xyz.reshape(B, NW, 1, MW * 3)
    q2 = new_xyz.reshape(B, 2, 1, (M // 2) * 3)
    idx = _bq_kernel(tab, q)
    out = _gather_kernel(tab, q2, idx)
    return out.reshape(B, OUTC, M, NS)


# speedup vs baseline: 1.7855x; 1.7855x over previous
import functools

import jax
import jax.numpy as jnp
from jax import lax
from jax.experimental import pallas as pl
from jax.experimental.pallas import tpu as pltpu
from jax.experimental.pallas import tpu_sc as plsc

B, N, M, NS, C = 8, 4096, 1024, 32, 128
RADIUS = 0.12
R2 = RADIUS * RADIUS

NC, NSUB, L = 2, 16, 16
NW = NC * NSUB
MW = M // NW
NG = MW // L
KC = 8
TC_ROWS = 136
NSLAB = 17
PU = 8
OUTC = 3 + C


_BCAST_DNUMS = lax.GatherDimensionNumbers(
    offset_dims=(), collapsed_slice_dims=(0,), start_index_map=(0,))


def _bcast(vec, j):
    return lax.gather(vec, jnp.full((L, 1), j, jnp.int32), _BCAST_DNUMS, (1,),
                      mode=lax.GatherScatterMode.PROMISE_IN_BOUNDS)


def _ball_query(pxyz_v, q_v, idx_v, lanes):
    zeros = jnp.zeros((L,), jnp.int32)
    qs = []
    for grp in range(NG):
        qsel = (grp * L + lanes) * 3
        qs.append((plsc.load_gather(q_v, [zeros, qsel]),
                   plsc.load_gather(q_v, [zeros, qsel + 1]),
                   plsc.load_gather(q_v, [zeros, qsel + 2]),
                   (grp * L + lanes) * NS))
        plsc.store_scatter(idx_v, [qs[grp][3]], zeros)

    def step(k, cnts):
        base = k * L
        basev = jnp.full((L,), base, jnp.int32)
        pxc = pxyz_v[0, pl.ds(base, L)]
        pyc = pxyz_v[1, pl.ds(base, L)]
        pzc = pxyz_v[2, pl.ds(base, L)]
        new = list(cnts)
        for j in range(L):
            px = _bcast(pxc, j)
            py = _bcast(pyc, j)
            pz = _bcast(pzc, j)
            pvec = basev + j
            for g in range(NG):
                qx, qy, qz, qoff = qs[g]
                cnt = new[g]
                dx = qx - px
                dy = qy - py
                dz = qz - pz
                d2 = dx * dx + dy * dy + dz * dz
                mask = d2 < R2
                wmask = mask & (cnt < NS)
                plsc.store_scatter(idx_v, [qoff + cnt], pvec, mask=wmask)
                new[g] = cnt + mask.astype(jnp.int32)
        return tuple(new)

    cnts = lax.fori_loop(0, N // L, step, (jnp.zeros((L,), jnp.int32),) * NG)

    for g in range(NG):
        qoff = qs[g][3]
        cnt = cnts[g]
        first = plsc.load_gather(idx_v, [qoff])
        for s in range(1, NS):
            cur = plsc.load_gather(idx_v, [qoff + s])
            sel = jnp.where(cnt > s, cur, first)
            plsc.store_scatter(idx_v, [qoff + s], sel)


def _body(tab_hbm, q_hbm, out_hbm, pxyz_v, q_v, idx_v, frows_v, obuf_v, sems):
    wid = lax.axis_index("s") * NC + lax.axis_index("c")
    lanes = lax.iota(jnp.int32, L)
    obase = wid * MW * NS

    def per_batch(b, _):
        pltpu.sync_copy(tab_hbm.at[b, pl.ds(0, 3), :], pxyz_v)
        pltpu.sync_copy(q_hbm.at[b, wid], q_v)

        _ball_query(pxyz_v, q_v, idx_v, lanes)

        def fin_copy(s, slot):
            return pltpu.make_async_copy(
                tab_hbm.at[b, pl.ds(s * KC, KC), :], frows_v.at[slot],
                sems.at[slot])

        def fout_copy(s, slot):
            nrow = KC if s < NSLAB - 1 else OUTC - KC * (NSLAB - 1)
            return pltpu.make_async_copy(
                obuf_v.at[slot, pl.ds(0, nrow)],
                out_hbm.at[b, pl.ds(s * KC, nrow), pl.ds(obase, MW * NS)],
                sems.at[2 + slot])

        fin_copy(0, 0).start()
        for s in range(NSLAB):
            slot = s % 2
            fin_copy(s, slot).wait()
            if s + 1 < NSLAB:
                fin_copy(s + 1, 1 - slot).start()
            if s >= 2:
                fout_copy(s - 2, slot).wait()

            def gath(t, _):
                iv = idx_v[pl.ds(t * L, L)]
                if s == 0:
                    mv3 = lax.shift_right_logical(t * L + lanes, 5) * 3
                for c in range(KC):
                    g = plsc.load_gather(frows_v,
                                         [jnp.full((L,), slot, jnp.int32),
                                          jnp.full((L,), c, jnp.int32), iv])
                    if s == 0 and c < 3:
                        qd = plsc.load_gather(
                            q_v, [jnp.zeros((L,), jnp.int32), mv3 + c])
                        g = g - qd
                    obuf_v[slot, c, pl.ds(t * L, L)] = g
                return 0

            lax.fori_loop(0, MW * NS // L, gath, 0)
            fout_copy(s, slot).start()
        fout_copy(NSLAB - 2, (NSLAB - 2) % 2).wait()
        fout_copy(NSLAB - 1, (NSLAB - 1) % 2).wait()
        return 0

    lax.fori_loop(0, B, per_batch, 0)


@functools.partial(
    pl.kernel,
    out_type=jax.ShapeDtypeStruct((B, OUTC, M * NS), jnp.float32),
    mesh=plsc.VectorSubcoreMesh(core_axis_name="c", subcore_axis_name="s"),
    scratch_types=[
        pltpu.VMEM((3, N), jnp.float32),
        pltpu.VMEM((1, MW * 3), jnp.float32),
        pltpu.VMEM((MW * NS,), jnp.int32),
        pltpu.VMEM((2, KC, N), jnp.float32),
        pltpu.VMEM((2, KC, MW * NS), jnp.float32),
        pltpu.SemaphoreType.DMA((4,)),
    ],
    compiler_params=pltpu.CompilerParams(needs_layout_passes=False),
)
def _qg_kernel(tab_hbm, q_hbm, out_hbm, *scratch):
    _body(tab_hbm, q_hbm, out_hbm, *scratch)


def kernel(xyz, new_xyz, features):
    xyz_t = jnp.transpose(xyz, (0, 2, 1))
    pad = jnp.zeros((B, TC_ROWS - 3 - C, N), jnp.float32)
    tab = jnp.concatenate([xyz_t, features, pad], axis=1)
    q = new_xyz.reshape(B, NW, 1, MW * 3)
    out = _qg_kernel(tab, q)
    return out.reshape(B, OUTC, M, NS)
